# baseline (speedup 1.0000x reference)
import functools

import jax
import jax.numpy as jnp
from jax import lax
from jax.experimental import pallas as pl
from jax.experimental.pallas import tpu as pltpu
from jax.experimental.pallas import tpu_sc as plsc

_NUM_CORES = 2
_NUM_SUBCORES = 16
_NUM_WORKERS = _NUM_CORES * _NUM_SUBCORES

_V_BLOCK = 2048
_NBUF = 4


_CHUNK = 128


def _sc_gather(emb_table, input_ids):
    batch, d_model = input_ids.shape[0], emb_table.shape[1]
    vocab = emb_table.shape[0]
    flat = emb_table.reshape(vocab * d_model)
    idx_flat = (
        input_ids[:, None] * d_model
        + jnp.arange(d_model, dtype=input_ids.dtype)[None, :]
    ).reshape(batch * d_model // _CHUNK, _CHUNK)
    elems_per_w = batch * d_model // _NUM_WORKERS
    chunks_per_w = elems_per_w // _CHUNK
    mesh = plsc.VectorSubcoreMesh(
        core_axis_name="c",
        subcore_axis_name="s",
        num_cores=_NUM_CORES,
        num_subcores=_NUM_SUBCORES,
    )

    @functools.partial(
        pl.kernel,
        mesh=mesh,
        out_type=jax.ShapeDtypeStruct((batch * d_model,), emb_table.dtype),
        scratch_types=[
            pltpu.VMEM((chunks_per_w, _CHUNK), jnp.int32),
            pltpu.VMEM((elems_per_w,), emb_table.dtype),
            pltpu.SemaphoreType.DMA,
        ],
        compiler_params=pltpu.CompilerParams(use_tc_tiling_on_sc=False),
    )
    def gather_kernel(flat_hbm, idx_hbm, out_hbm, idx_v, gath_v, sem):
        wid = lax.axis_index("s") * _NUM_CORES + lax.axis_index("c")
        pltpu.sync_copy(idx_hbm.at[pl.ds(wid * chunks_per_w, chunks_per_w)], idx_v)
        copies = [
            pltpu.async_copy(
                flat_hbm.at[idx_v.at[c]],
                gath_v.at[pl.ds(c * _CHUNK, _CHUNK)],
                sem,
            )
            for c in range(chunks_per_w)
        ]
        for cp in copies:
            cp.wait()
        pltpu.sync_copy(gath_v, out_hbm.at[pl.ds(wid * elems_per_w, elems_per_w)])

    return gather_kernel(flat, idx_flat).reshape(batch, d_model)


def _make_matmul_body(nsteps, v_tail, batch):
    def body(x_ref, w_ref, b_ref, out_ref, acc_ref, sem_ref):
        j = pl.program_id(0)
        slot = lax.rem(j, _NBUF)

        @pl.when(j >= _NBUF)
        def _():
            pltpu.make_async_copy(
                acc_ref.at[slot],
                out_ref.at[pl.ds((j - _NBUF) * _V_BLOCK, _V_BLOCK), :],
                sem_ref.at[slot],
            ).wait()

        acc_ref[slot] = lax.dot_general(
            w_ref[...],
            x_ref[...],
            (((0,), (1,)), ((), ())),
            preferred_element_type=jnp.float32,
        ) + lax.dot_general(
            b_ref[...],
            jnp.ones((1, batch), jnp.float32),
            (((0,), (0,)), ((), ())),
            preferred_element_type=jnp.float32,
        )

        @pl.when(j < nsteps - 1)
        def _():
            pltpu.make_async_copy(
                acc_ref.at[slot],
                out_ref.at[pl.ds(j * _V_BLOCK, _V_BLOCK), :],
                sem_ref.at[slot],
            ).start()

        @pl.when(j == nsteps - 1)
        def _():
            last = nsteps - 1
            last_slot = last % _NBUF
            pltpu.make_async_copy(
                acc_ref.at[last_slot, :v_tail],
                out_ref.at[pl.ds(last * _V_BLOCK, v_tail), :],
                sem_ref.at[last_slot],
            ).start()
            for k in range(max(0, nsteps - _NBUF), last):
                s = k % _NBUF
                pltpu.make_async_copy(
                    acc_ref.at[s],
                    out_ref.at[pl.ds(k * _V_BLOCK, _V_BLOCK), :],
                    sem_ref.at[s],
                ).wait()
            pltpu.make_async_copy(
                acc_ref.at[last_slot, :v_tail],
                out_ref.at[pl.ds(last * _V_BLOCK, v_tail), :],
                sem_ref.at[last_slot],
            ).wait()

    return body


def _tc_project(x, w, b):
    batch, d_model = x.shape
    vocab = w.shape[1]
    nsteps = pl.cdiv(vocab, _V_BLOCK)
    v_tail = vocab - (nsteps - 1) * _V_BLOCK
    b2d = b.reshape(1, vocab)
    return pl.pallas_call(
        _make_matmul_body(nsteps, v_tail, batch),
        grid=(nsteps,),
        in_specs=[
            pl.BlockSpec((batch, d_model), lambda j: (0, 0)),
            pl.BlockSpec((d_model, _V_BLOCK), lambda j: (0, j)),
            pl.BlockSpec((1, _V_BLOCK), lambda j: (0, j)),
        ],
        out_specs=pl.BlockSpec(memory_space=pl.ANY),
        out_shape=jax.ShapeDtypeStruct((vocab, batch), jnp.float32),
        scratch_shapes=[
            pltpu.VMEM((_NBUF, _V_BLOCK, batch), jnp.float32),
            pltpu.SemaphoreType.DMA((_NBUF,)),
        ],
    )(x, w, b2d)


def kernel(input_ids, emb_table, W, b):
    x = _sc_gather(emb_table, input_ids.astype(jnp.int32))
    logits_t = _tc_project(x, W, b)
    return logits_t.T


# speedup vs baseline: 1.0175x; 1.0175x over previous
import functools

import jax
import jax.numpy as jnp
from jax import lax
from jax.experimental import pallas as pl
from jax.experimental.pallas import tpu as pltpu
from jax.experimental.pallas import tpu_sc as plsc

_NUM_CORES = 2
_NUM_SUBCORES = 16
_NUM_WORKERS = _NUM_CORES * _NUM_SUBCORES

_V_BLOCK = 2048
_NBUF = 4


def _sc_gather(emb_table, input_ids):
    batch, d_model = input_ids.shape[0], emb_table.shape[1]
    b_per_w = batch // _NUM_WORKERS
    mesh = plsc.VectorSubcoreMesh(
        core_axis_name="c",
        subcore_axis_name="s",
        num_cores=_NUM_CORES,
        num_subcores=_NUM_SUBCORES,
    )

    @functools.partial(
        pl.kernel,
        mesh=mesh,
        out_type=jax.ShapeDtypeStruct((batch, d_model), emb_table.dtype),
        scratch_types=[
            pltpu.VMEM((b_per_w,), jnp.int32),
            pltpu.VMEM((b_per_w, d_model), emb_table.dtype),
            pltpu.SemaphoreType.DMA,
        ],
        compiler_params=pltpu.CompilerParams(use_tc_tiling_on_sc=False),
    )
    def gather_kernel(table_hbm, idx_hbm, out_hbm, idx_v, rows_v, sem):
        wid = lax.axis_index("s") * _NUM_CORES + lax.axis_index("c")
        base = wid * b_per_w
        pltpu.sync_copy(idx_hbm.at[pl.ds(base, b_per_w)], idx_v)
        pltpu.async_copy(table_hbm.at[idx_v], rows_v, sem).wait()
        pltpu.sync_copy(rows_v, out_hbm.at[pl.ds(base, b_per_w)])

    return gather_kernel(emb_table, input_ids)


def _make_matmul_body(nsteps, v_tail, batch):
    def body(x_ref, w_ref, b_ref, out_ref, acc_ref, sem_ref):
        j = pl.program_id(0)
        slot = lax.rem(j, _NBUF)

        @pl.when(j >= _NBUF)
        def _():
            pltpu.make_async_copy(
                acc_ref.at[slot],
                out_ref.at[pl.ds((j - _NBUF) * _V_BLOCK, _V_BLOCK), :],
                sem_ref.at[slot],
            ).wait()

        acc_ref[slot] = lax.dot_general(
            w_ref[...],
            x_ref[...],
            (((0,), (1,)), ((), ())),
            preferred_element_type=jnp.float32,
        ) + lax.dot_general(
            b_ref[...],
            jnp.ones((1, batch), jnp.float32),
            (((0,), (0,)), ((), ())),
            preferred_element_type=jnp.float32,
        )

        @pl.when(j < nsteps - 1)
        def _():
            pltpu.make_async_copy(
                acc_ref.at[slot],
                out_ref.at[pl.ds(j * _V_BLOCK, _V_BLOCK), :],
                sem_ref.at[slot],
            ).start()

        @pl.when(j == nsteps - 1)
        def _():
            last = nsteps - 1
            last_slot = last % _NBUF
            pltpu.make_async_copy(
                acc_ref.at[last_slot, :v_tail],
                out_ref.at[pl.ds(last * _V_BLOCK, v_tail), :],
                sem_ref.at[last_slot],
            ).start()
            for k in range(max(0, nsteps - _NBUF), last):
                s = k % _NBUF
                pltpu.make_async_copy(
                    acc_ref.at[s],
                    out_ref.at[pl.ds(k * _V_BLOCK, _V_BLOCK), :],
                    sem_ref.at[s],
                ).wait()
            pltpu.make_async_copy(
                acc_ref.at[last_slot, :v_tail],
                out_ref.at[pl.ds(last * _V_BLOCK, v_tail), :],
                sem_ref.at[last_slot],
            ).wait()

    return body


def _tc_project(x, w, b):
    batch, d_model = x.shape
    vocab = w.shape[1]
    nsteps = pl.cdiv(vocab, _V_BLOCK)
    v_tail = vocab - (nsteps - 1) * _V_BLOCK
    b2d = b.reshape(1, vocab)
    return pl.pallas_call(
        _make_matmul_body(nsteps, v_tail, batch),
        grid=(nsteps,),
        in_specs=[
            pl.BlockSpec((batch, d_model), lambda j: (0, 0)),
            pl.BlockSpec((d_model, _V_BLOCK), lambda j: (0, j)),
            pl.BlockSpec((1, _V_BLOCK), lambda j: (0, j)),
        ],
        out_specs=pl.BlockSpec(memory_space=pl.ANY),
        out_shape=jax.ShapeDtypeStruct((vocab, batch), jnp.float32),
        scratch_shapes=[
            pltpu.VMEM((_NBUF, _V_BLOCK, batch), jnp.float32),
            pltpu.SemaphoreType.DMA((_NBUF,)),
        ],
    )(x, w, b2d)


def kernel(input_ids, emb_table, W, b):
    x = _sc_gather(emb_table, input_ids.astype(jnp.int32))
    logits_t = _tc_project(x, W, b)
    return logits_t.T


# speedup vs baseline: 1.0206x; 1.0030x over previous
import functools

import jax
import jax.numpy as jnp
from jax import lax
from jax.experimental import pallas as pl
from jax.experimental.pallas import tpu as pltpu
from jax.experimental.pallas import tpu_sc as plsc

_NUM_CORES = 2
_NUM_SUBCORES = 16
_NUM_WORKERS = _NUM_CORES * _NUM_SUBCORES

_V_BLOCK = 2048
_NBUF = 6


def _sc_gather(emb_table, input_ids):
    batch, d_model = input_ids.shape[0], emb_table.shape[1]
    b_per_w = batch // _NUM_WORKERS
    mesh = plsc.VectorSubcoreMesh(
        core_axis_name="c",
        subcore_axis_name="s",
        num_cores=_NUM_CORES,
        num_subcores=_NUM_SUBCORES,
    )

    @functools.partial(
        pl.kernel,
        mesh=mesh,
        out_type=jax.ShapeDtypeStruct((batch, d_model), emb_table.dtype),
        scratch_types=[
            pltpu.VMEM((b_per_w,), jnp.int32),
            pltpu.VMEM((b_per_w, d_model), emb_table.dtype),
            pltpu.SemaphoreType.DMA,
        ],
        compiler_params=pltpu.CompilerParams(use_tc_tiling_on_sc=False),
    )
    def gather_kernel(table_hbm, idx_hbm, out_hbm, idx_v, rows_v, sem):
        wid = lax.axis_index("s") * _NUM_CORES + lax.axis_index("c")
        base = wid * b_per_w
        pltpu.sync_copy(idx_hbm.at[pl.ds(base, b_per_w)], idx_v)
        pltpu.async_copy(table_hbm.at[idx_v], rows_v, sem).wait()
        pltpu.sync_copy(rows_v, out_hbm.at[pl.ds(base, b_per_w)])

    return gather_kernel(emb_table, input_ids)


def _make_matmul_body(nsteps, v_tail, batch):
    def body(x_ref, w_ref, b_ref, out_ref, acc_ref, sem_ref):
        j = pl.program_id(0)
        slot = lax.rem(j, _NBUF)

        @pl.when(j >= _NBUF)
        def _():
            pltpu.make_async_copy(
                acc_ref.at[slot],
                out_ref.at[pl.ds((j - _NBUF) * _V_BLOCK, _V_BLOCK), :],
                sem_ref.at[slot],
            ).wait()

        acc_ref[slot] = lax.dot_general(
            w_ref[...],
            x_ref[...],
            (((0,), (1,)), ((), ())),
            preferred_element_type=jnp.float32,
        ) + lax.dot_general(
            b_ref[...],
            jnp.ones((1, batch), jnp.float32),
            (((0,), (0,)), ((), ())),
            preferred_element_type=jnp.float32,
        )

        @pl.when(j < nsteps - 1)
        def _():
            pltpu.make_async_copy(
                acc_ref.at[slot],
                out_ref.at[pl.ds(j * _V_BLOCK, _V_BLOCK), :],
                sem_ref.at[slot],
            ).start()

        @pl.when(j == nsteps - 1)
        def _():
            last = nsteps - 1
            last_slot = last % _NBUF
            pltpu.make_async_copy(
                acc_ref.at[last_slot, :v_tail],
                out_ref.at[pl.ds(last * _V_BLOCK, v_tail), :],
                sem_ref.at[last_slot],
            ).start()
            for k in range(max(0, nsteps - _NBUF), last):
                s = k % _NBUF
                pltpu.make_async_copy(
                    acc_ref.at[s],
                    out_ref.at[pl.ds(k * _V_BLOCK, _V_BLOCK), :],
                    sem_ref.at[s],
                ).wait()
            pltpu.make_async_copy(
                acc_ref.at[last_slot, :v_tail],
                out_ref.at[pl.ds(last * _V_BLOCK, v_tail), :],
                sem_ref.at[last_slot],
            ).wait()

    return body


def _tc_project(x, w, b):
    batch, d_model = x.shape
    vocab = w.shape[1]
    nsteps = pl.cdiv(vocab, _V_BLOCK)
    v_tail = vocab - (nsteps - 1) * _V_BLOCK
    b2d = b.reshape(1, vocab)
    return pl.pallas_call(
        _make_matmul_body(nsteps, v_tail, batch),
        grid=(nsteps,),
        in_specs=[
            pl.BlockSpec((batch, d_model), lambda j: (0, 0)),
            pl.BlockSpec((d_model, _V_BLOCK), lambda j: (0, j)),
            pl.BlockSpec((1, _V_BLOCK), lambda j: (0, j)),
        ],
        out_specs=pl.BlockSpec(memory_space=pl.ANY),
        out_shape=jax.ShapeDtypeStruct((vocab, batch), jnp.float32),
        scratch_shapes=[
            pltpu.VMEM((_NBUF, _V_BLOCK, batch), jnp.float32),
            pltpu.SemaphoreType.DMA((_NBUF,)),
        ],
    )(x, w, b2d)


def kernel(input_ids, emb_table, W, b):
    x = _sc_gather(emb_table, input_ids.astype(jnp.int32))
    logits_t = _tc_project(x, W, b)
    return logits_t.T
